# baseline (speedup 1.0000x reference)
import functools

import jax
import jax.numpy as jnp
from jax import lax
from jax.experimental import pallas as pl
from jax.experimental.pallas import tpu as pltpu
from jax.experimental.pallas import tpu_sc as plsc

_EMB = 128
_K = 32
_STEP = 256
_NSTEP = 4
_PACK = 128 // _K
_NROWS = _NSTEP * _K + 16


def _lane_gather(vec, idx):
    return lax.gather(
        vec, idx[:, None],
        lax.GatherDimensionNumbers(
            offset_dims=(), collapsed_slice_dims=(0,), start_index_map=(0,)),
        (1,), mode=lax.GatherScatterMode.PROMISE_IN_BOUNDS)


def _eq_mask(a, b):
    d = a - b
    return 1 + ((d | -d) >> 31)


def _sc_gather(node_emb, neighbors2d, path_i32):
    mesh = plsc.VectorSubcoreMesh(core_axis_name="c", subcore_axis_name="s",
                                  num_cores=1)

    @functools.partial(
        pl.kernel,
        out_type=jax.ShapeDtypeStruct((_NROWS, _EMB), jnp.float32),
        mesh=mesh,
        scratch_types=[
            pltpu.VMEM((16,), jnp.int32),
            pltpu.VMEM((16, _EMB), jnp.int32),
            pltpu.VMEM((16, _EMB), jnp.float32),
            pltpu.SemaphoreType.DMA,
            pltpu.SemaphoreType.DMA,
        ],
    )
    def gather_kernel(emb_hbm, nbrtab_hbm, path_hbm, out_hbm,
                      path_v, nbrrows_v, emb_v, sem_idx, sem_emb):
        wid = lax.axis_index("c") * 16 + lax.axis_index("s")

        @pl.when(wid <= 2 * _NSTEP)
        def _():
            pltpu.sync_copy(path_hbm.at[pl.ds(0, 8)], path_v.at[pl.ds(0, 8)])
            pltpu.sync_copy(path_hbm.at[pl.ds(8, 1)], path_v.at[pl.ds(8, 1)])
            lanes = lax.iota(jnp.int32, 16)
            roots = _lane_gather(path_v[...], jnp.minimum(lanes * 2 + 2, 8))
            w = jnp.minimum(wid >> 1, _NSTEP - 1)
            h = wid & 1
            rw = _lane_gather(roots, jnp.full((16,), w, jnp.int32))
            pltpu.async_copy(nbrtab_hbm.at[rw >> 2], nbrrows_v,
                             sem_idx).wait()
            bb = ((rw & (_PACK - 1)) << 1) + h
            ids = jnp.zeros((16,), jnp.int32)
            for t in range(8):
                ids = ids + nbrrows_v[0, pl.ds(16 * t, 16)] * _eq_mask(bb, t)
            m8 = _eq_mask(wid, 2 * _NSTEP)
            ids = ids * (1 - m8) + roots * m8
            pltpu.async_copy(emb_hbm.at[ids], emb_v, sem_emb).wait()
            pltpu.sync_copy(emb_v, out_hbm.at[pl.ds(wid * 16, 16)])

    return gather_kernel(node_emb, neighbors2d, path_i32)


def _dense_body(g_ref, wp_ref, bp_ref, wk_ref, wr_ref, bl_ref,
                w1_ref, b1_ref, w2_ref, b2_ref, w3_ref, b3_ref, out_ref):
    f32 = jnp.float32

    def dot(a, b):
        return lax.dot_general(a, b, (((1,), (0,)), ((), ())),
                               preferred_element_type=f32,
                               precision=lax.Precision.HIGHEST)

    wp = wp_ref[...]
    nbr_p = dot(g_ref[:_NSTEP * _K, :], wp[_EMB:])
    self_p = dot(g_ref[_NSTEP * _K:_NSTEP * _K + 8, :], wp[:_EMB])
    bp = bp_ref[...][None, :]

    sfs = []
    for w in range(_NSTEP):
        blk = nbr_p[_K * w:_K * (w + 1)] + self_p[w][None, :] + bp
        blk = jnp.maximum(blk, 0.0)
        sfs.append(jnp.max(blk, axis=0, keepdims=True))
    sf = jnp.concatenate(sfs, axis=0)

    pre = dot(sf, wk_ref[...]) + bl_ref[...][None, :]
    wr = wr_ref[...]
    h = jnp.zeros((1, _STEP), f32)
    c = jnp.zeros((1, _STEP), f32)
    for i in range(_NSTEP):
        z = pre[i:i + 1] + dot(h, wr)
        zi = z[:, :_STEP]
        zf = z[:, _STEP:2 * _STEP]
        zc = z[:, 2 * _STEP:3 * _STEP]
        zo = z[:, 3 * _STEP:]
        c = jax.nn.sigmoid(zf) * c + jax.nn.sigmoid(zi) * jnp.tanh(zc)
        h = jax.nn.sigmoid(zo) * jnp.tanh(c)

    h1 = jnp.maximum(dot(h, w1_ref[...]) + b1_ref[...][None, :], 0.0)
    h2 = jnp.maximum(dot(h1, w2_ref[...]) + b2_ref[...][None, :], 0.0)
    logits = dot(h2, w3_ref[...]) + b3_ref[...][None, :]
    out_ref[...] = jax.nn.softmax(logits, axis=-1)[0]


def _tc_dense(gathered, W_pool, b_pool, Wk, Wr, b_lstm,
              W1, b1, W2, b2, W3, b3):
    return pl.pallas_call(
        _dense_body,
        out_shape=jax.ShapeDtypeStruct((2,), jnp.float32),
    )(gathered, W_pool, b_pool, Wk, Wr, b_lstm, W1, b1, W2, b2, W3, b3)


def kernel(node_emb, neighbors, path, W_pool, b_pool, Wk, Wr, b_lstm,
           W1, b1, W2, b2, W3, b3):
    nbr2d = neighbors.astype(jnp.int32).reshape(-1, _EMB)
    gathered = _sc_gather(node_emb, nbr2d, path.astype(jnp.int32))
    return _tc_dense(gathered, W_pool, b_pool, Wk, Wr, b_lstm,
                     W1, b1, W2, b2, W3, b3)


# speedup vs baseline: 1.0046x; 1.0046x over previous
import functools

import jax
import jax.numpy as jnp
from jax import lax
from jax.experimental import pallas as pl
from jax.experimental.pallas import tpu as pltpu
from jax.experimental.pallas import tpu_sc as plsc

_EMB = 128
_K = 32
_STEP = 256
_NSTEP = 4
_PACK = 128 // _K
_NROWS = _NSTEP * _K + 16


def _lane_gather(vec, idx):
    return lax.gather(
        vec, idx[:, None],
        lax.GatherDimensionNumbers(
            offset_dims=(), collapsed_slice_dims=(0,), start_index_map=(0,)),
        (1,), mode=lax.GatherScatterMode.PROMISE_IN_BOUNDS)


def _eq_mask(a, b):
    d = a - b
    return 1 + ((d | -d) >> 31)


def _sc_gather(node_emb, neighbors2d, path_i32):
    mesh = plsc.VectorSubcoreMesh(core_axis_name="c", subcore_axis_name="s",
                                  num_cores=1)

    @functools.partial(
        pl.kernel,
        out_type=jax.ShapeDtypeStruct((_NROWS, _EMB), jnp.float32),
        mesh=mesh,
        scratch_types=[
            pltpu.VMEM((16,), jnp.int32),
            pltpu.VMEM((16, _EMB), jnp.int32),
            pltpu.VMEM((16, _EMB), jnp.float32),
            pltpu.SemaphoreType.DMA,
            pltpu.SemaphoreType.DMA,
        ],
    )
    def gather_kernel(emb_hbm, nbrtab_hbm, path_hbm, out_hbm,
                      path_v, nbrrows_v, emb_v, sem_idx, sem_emb):
        wid = lax.axis_index("c") * 16 + lax.axis_index("s")

        @pl.when(wid <= 2 * _NSTEP)
        def _():
            pltpu.sync_copy(path_hbm.at[pl.ds(0, 8)], path_v.at[pl.ds(0, 8)])
            pltpu.sync_copy(path_hbm.at[pl.ds(8, 1)], path_v.at[pl.ds(8, 1)])
            lanes = lax.iota(jnp.int32, 16)
            roots = _lane_gather(path_v[...], jnp.minimum(lanes * 2 + 2, 8))
            w = jnp.minimum(wid >> 1, _NSTEP - 1)
            h = wid & 1
            rw = _lane_gather(roots, jnp.full((16,), w, jnp.int32))
            pltpu.async_copy(nbrtab_hbm.at[rw >> 2], nbrrows_v,
                             sem_idx).wait()
            bb = ((rw & (_PACK - 1)) << 1) + h
            ids = jnp.zeros((16,), jnp.int32)
            for t in range(8):
                ids = ids + nbrrows_v[0, pl.ds(16 * t, 16)] * _eq_mask(bb, t)
            m8 = _eq_mask(wid, 2 * _NSTEP)
            ids = ids * (1 - m8) + roots * m8
            pltpu.async_copy(emb_hbm.at[ids], emb_v, sem_emb).wait()
            pltpu.sync_copy(emb_v, out_hbm.at[pl.ds(wid * 16, 16)])

    return gather_kernel(node_emb, neighbors2d, path_i32)


def _dense_body(g_hbm, wp_hbm, bp_hbm, wk_hbm, wr_hbm, bl_hbm,
                w1_hbm, b1_hbm, w2_hbm, b2_hbm, w3_hbm, b3_hbm, out_ref,
                g_v, wp_v, bp_v, wk_v, wr_v, bl_v, w1_v, b1_v, w2_v, b2_v,
                w3_v, b3_v, sem_a, sem_k, sem_r, sem_1, sem_2, sem_3):
    f32 = jnp.float32

    def dot(a, b):
        return lax.dot_general(a, b, (((1,), (0,)), ((), ())),
                               preferred_element_type=f32,
                               precision=lax.Precision.HIGHEST)

    cp = pltpu.make_async_copy
    cps_a = [cp(g_hbm, g_v, sem_a), cp(wp_hbm, wp_v, sem_a),
             cp(bp_hbm, bp_v, sem_a)]
    for c_ in cps_a:
        c_.start()
    cps_k = [cp(wk_hbm, wk_v, sem_k), cp(bl_hbm, bl_v, sem_k)]
    for c_ in cps_k:
        c_.start()
    cp_r = cp(wr_hbm, wr_v, sem_r)
    cp_r.start()
    cps_1 = [cp(w1_hbm, w1_v, sem_1), cp(b1_hbm, b1_v, sem_1)]
    for c_ in cps_1:
        c_.start()
    cps_2 = [cp(w2_hbm, w2_v, sem_2), cp(b2_hbm, b2_v, sem_2)]
    for c_ in cps_2:
        c_.start()
    cps_3 = [cp(w3_hbm, w3_v, sem_3), cp(b3_hbm, b3_v, sem_3)]
    for c_ in cps_3:
        c_.start()

    for c_ in cps_a:
        c_.wait()
    wp = wp_v[...]
    nbr_p = dot(g_v[:_NSTEP * _K, :], wp[_EMB:])
    self_p = dot(g_v[_NSTEP * _K:_NSTEP * _K + 8, :], wp[:_EMB])
    bp = bp_v[...][None, :]

    sfs = []
    for w in range(_NSTEP):
        blk = nbr_p[_K * w:_K * (w + 1)] + self_p[w][None, :] + bp
        blk = jnp.maximum(blk, 0.0)
        sfs.append(jnp.max(blk, axis=0, keepdims=True))
    sf = jnp.concatenate(sfs, axis=0)

    for c_ in cps_k:
        c_.wait()
    pre = dot(sf, wk_v[...]) + bl_v[...][None, :]
    cp_r.wait()
    wr = wr_v[...]
    h = jnp.zeros((1, _STEP), f32)
    c = jnp.zeros((1, _STEP), f32)
    for i in range(_NSTEP):
        z = pre[i:i + 1] + dot(h, wr)
        zi = z[:, :_STEP]
        zf = z[:, _STEP:2 * _STEP]
        zc = z[:, 2 * _STEP:3 * _STEP]
        zo = z[:, 3 * _STEP:]
        c = jax.nn.sigmoid(zf) * c + jax.nn.sigmoid(zi) * jnp.tanh(zc)
        h = jax.nn.sigmoid(zo) * jnp.tanh(c)

    for c_ in cps_1:
        c_.wait()
    h1 = jnp.maximum(dot(h, w1_v[...]) + b1_v[...][None, :], 0.0)
    for c_ in cps_2:
        c_.wait()
    h2 = jnp.maximum(dot(h1, w2_v[...]) + b2_v[...][None, :], 0.0)
    for c_ in cps_3:
        c_.wait()
    logits = dot(h2, w3_v[...]) + b3_v[...][None, :]
    out_ref[...] = jax.nn.softmax(logits, axis=-1)[0]


def _tc_dense(gathered, W_pool, b_pool, Wk, Wr, b_lstm,
              W1, b1, W2, b2, W3, b3):
    any_spec = pl.BlockSpec(memory_space=pl.ANY)
    return pl.pallas_call(
        _dense_body,
        in_specs=[any_spec] * 12,
        out_shape=jax.ShapeDtypeStruct((2,), jnp.float32),
        scratch_shapes=[
            pltpu.VMEM((_NROWS, _EMB), jnp.float32),
            pltpu.VMEM((2 * _EMB, _STEP), jnp.float32),
            pltpu.VMEM((_STEP,), jnp.float32),
            pltpu.VMEM((_STEP, 4 * _STEP), jnp.float32),
            pltpu.VMEM((_STEP, 4 * _STEP), jnp.float32),
            pltpu.VMEM((4 * _STEP,), jnp.float32),
            pltpu.VMEM((_STEP, _STEP), jnp.float32),
            pltpu.VMEM((_STEP,), jnp.float32),
            pltpu.VMEM((_STEP, _STEP), jnp.float32),
            pltpu.VMEM((_STEP,), jnp.float32),
            pltpu.VMEM((_STEP, 2), jnp.float32),
            pltpu.VMEM((2,), jnp.float32),
            pltpu.SemaphoreType.DMA,
            pltpu.SemaphoreType.DMA,
            pltpu.SemaphoreType.DMA,
            pltpu.SemaphoreType.DMA,
            pltpu.SemaphoreType.DMA,
            pltpu.SemaphoreType.DMA,
        ],
    )(gathered, W_pool, b_pool, Wk, Wr, b_lstm, W1, b1, W2, b2, W3, b3)


def kernel(node_emb, neighbors, path, W_pool, b_pool, Wk, Wr, b_lstm,
           W1, b1, W2, b2, W3, b3):
    nbr2d = neighbors.astype(jnp.int32).reshape(-1, _EMB)
    gathered = _sc_gather(node_emb, nbr2d, path.astype(jnp.int32))
    return _tc_dense(gathered, W_pool, b_pool, Wk, Wr, b_lstm,
                     W1, b1, W2, b2, W3, b3)


# speedup vs baseline: 1.0862x; 1.0813x over previous
import functools

import jax
import jax.numpy as jnp
from jax import lax
from jax.experimental import pallas as pl
from jax.experimental.pallas import tpu as pltpu
from jax.experimental.pallas import tpu_sc as plsc

_EMB = 128
_K = 32
_STEP = 256
_NSTEP = 4
_PACK = 128 // _K
_NROWS = _NSTEP * _K + 16


def _lane_gather(vec, idx):
    return lax.gather(
        vec, idx[:, None],
        lax.GatherDimensionNumbers(
            offset_dims=(), collapsed_slice_dims=(0,), start_index_map=(0,)),
        (1,), mode=lax.GatherScatterMode.PROMISE_IN_BOUNDS)


def _eq_mask(a, b):
    d = a - b
    return 1 + ((d | -d) >> 31)


def _sc_gather(node_emb, neighbors2d, path_i32):
    mesh = plsc.VectorSubcoreMesh(core_axis_name="c", subcore_axis_name="s",
                                  num_cores=1)

    @functools.partial(
        pl.kernel,
        out_type=jax.ShapeDtypeStruct((_NROWS, _EMB), jnp.float32),
        mesh=mesh,
        scratch_types=[
            pltpu.VMEM((16,), jnp.int32),
            pltpu.VMEM((16, _EMB), jnp.int32),
            pltpu.VMEM((16, _EMB), jnp.float32),
            pltpu.SemaphoreType.DMA,
            pltpu.SemaphoreType.DMA,
        ],
    )
    def gather_kernel(emb_hbm, nbrtab_hbm, path_hbm, out_hbm,
                      path_v, nbrrows_v, emb_v, sem_idx, sem_emb):
        wid = lax.axis_index("c") * 16 + lax.axis_index("s")

        @pl.when(wid <= 2 * _NSTEP)
        def _():
            pltpu.sync_copy(path_hbm.at[pl.ds(0, 8)], path_v.at[pl.ds(0, 8)])
            pltpu.sync_copy(path_hbm.at[pl.ds(8, 1)], path_v.at[pl.ds(8, 1)])
            lanes = lax.iota(jnp.int32, 16)
            roots = _lane_gather(path_v[...], jnp.minimum(lanes * 2 + 2, 8))
            w = jnp.minimum(wid >> 1, _NSTEP - 1)
            h = wid & 1
            rw = _lane_gather(roots, jnp.full((16,), w, jnp.int32))
            pltpu.async_copy(nbrtab_hbm.at[rw >> 2], nbrrows_v,
                             sem_idx).wait()
            bb = ((rw & (_PACK - 1)) << 1) + h
            ids = jnp.zeros((16,), jnp.int32)
            for t in range(8):
                ids = ids + nbrrows_v[0, pl.ds(16 * t, 16)] * _eq_mask(bb, t)
            m8 = _eq_mask(wid, 2 * _NSTEP)
            ids = ids * (1 - m8) + roots * m8
            pltpu.async_copy(emb_hbm.at[ids], emb_v, sem_emb).wait()
            pltpu.sync_copy(emb_v, out_hbm.at[pl.ds(wid * 16, 16)])

    return gather_kernel(node_emb, neighbors2d, path_i32)


def _dense_body(g_hbm, wp_hbm, bp_hbm, wk_hbm, wr_hbm, bl_hbm,
                w1_hbm, b1_hbm, w2_hbm, b2_hbm, w3_hbm, b3_hbm, out_ref,
                g_v, wp_v, bp_v, wk_v, wr_v, bl_v, w1_v, b1_v, w2_v, b2_v,
                w3_v, b3_v, sem_a, sem_k, sem_r, sem_1, sem_2, sem_3):
    f32 = jnp.float32

    def dot(a, b):
        return lax.dot_general(a, b, (((1,), (0,)), ((), ())),
                               preferred_element_type=f32,
                               precision=lax.Precision.DEFAULT)

    cp = pltpu.make_async_copy
    cps_a = [cp(g_hbm, g_v, sem_a), cp(wp_hbm, wp_v, sem_a),
             cp(bp_hbm, bp_v, sem_a)]
    for c_ in cps_a:
        c_.start()
    cps_k = [cp(wk_hbm, wk_v, sem_k), cp(bl_hbm, bl_v, sem_k)]
    for c_ in cps_k:
        c_.start()
    cp_r = cp(wr_hbm, wr_v, sem_r)
    cp_r.start()
    cps_1 = [cp(w1_hbm, w1_v, sem_1), cp(b1_hbm, b1_v, sem_1)]
    for c_ in cps_1:
        c_.start()
    cps_2 = [cp(w2_hbm, w2_v, sem_2), cp(b2_hbm, b2_v, sem_2)]
    for c_ in cps_2:
        c_.start()
    cps_3 = [cp(w3_hbm, w3_v, sem_3), cp(b3_hbm, b3_v, sem_3)]
    for c_ in cps_3:
        c_.start()

    for c_ in cps_a:
        c_.wait()
    wp = wp_v[...]
    nbr_p = dot(g_v[:_NSTEP * _K, :], wp[_EMB:])
    self_p = dot(g_v[_NSTEP * _K:_NSTEP * _K + 8, :], wp[:_EMB])
    bp = bp_v[...][None, :]

    sfs = []
    for w in range(_NSTEP):
        blk = nbr_p[_K * w:_K * (w + 1)] + self_p[w][None, :] + bp
        blk = jnp.maximum(blk, 0.0)
        sfs.append(jnp.max(blk, axis=0, keepdims=True))
    sf = jnp.concatenate(sfs, axis=0)

    for c_ in cps_k:
        c_.wait()
    pre = dot(sf, wk_v[...]) + bl_v[...][None, :]
    cp_r.wait()
    wr = wr_v[...]
    h = jnp.zeros((1, _STEP), f32)
    c = jnp.zeros((1, _STEP), f32)
    for i in range(_NSTEP):
        z = pre[i:i + 1] + dot(h, wr)
        zi = z[:, :_STEP]
        zf = z[:, _STEP:2 * _STEP]
        zc = z[:, 2 * _STEP:3 * _STEP]
        zo = z[:, 3 * _STEP:]
        c = jax.nn.sigmoid(zf) * c + jax.nn.sigmoid(zi) * jnp.tanh(zc)
        h = jax.nn.sigmoid(zo) * jnp.tanh(c)

    for c_ in cps_1:
        c_.wait()
    h1 = jnp.maximum(dot(h, w1_v[...]) + b1_v[...][None, :], 0.0)
    for c_ in cps_2:
        c_.wait()
    h2 = jnp.maximum(dot(h1, w2_v[...]) + b2_v[...][None, :], 0.0)
    for c_ in cps_3:
        c_.wait()
    logits = dot(h2, w3_v[...]) + b3_v[...][None, :]
    out_ref[...] = jax.nn.softmax(logits, axis=-1)[0]


def _tc_dense(gathered, W_pool, b_pool, Wk, Wr, b_lstm,
              W1, b1, W2, b2, W3, b3):
    any_spec = pl.BlockSpec(memory_space=pl.ANY)
    return pl.pallas_call(
        _dense_body,
        in_specs=[any_spec] * 12,
        out_shape=jax.ShapeDtypeStruct((2,), jnp.float32),
        scratch_shapes=[
            pltpu.VMEM((_NROWS, _EMB), jnp.float32),
            pltpu.VMEM((2 * _EMB, _STEP), jnp.float32),
            pltpu.VMEM((_STEP,), jnp.float32),
            pltpu.VMEM((_STEP, 4 * _STEP), jnp.float32),
            pltpu.VMEM((_STEP, 4 * _STEP), jnp.float32),
            pltpu.VMEM((4 * _STEP,), jnp.float32),
            pltpu.VMEM((_STEP, _STEP), jnp.float32),
            pltpu.VMEM((_STEP,), jnp.float32),
            pltpu.VMEM((_STEP, _STEP), jnp.float32),
            pltpu.VMEM((_STEP,), jnp.float32),
            pltpu.VMEM((_STEP, 2), jnp.float32),
            pltpu.VMEM((2,), jnp.float32),
            pltpu.SemaphoreType.DMA,
            pltpu.SemaphoreType.DMA,
            pltpu.SemaphoreType.DMA,
            pltpu.SemaphoreType.DMA,
            pltpu.SemaphoreType.DMA,
            pltpu.SemaphoreType.DMA,
        ],
    )(gathered, W_pool, b_pool, Wk, Wr, b_lstm, W1, b1, W2, b2, W3, b3)


def kernel(node_emb, neighbors, path, W_pool, b_pool, Wk, Wr, b_lstm,
           W1, b1, W2, b2, W3, b3):
    nbr2d = neighbors.astype(jnp.int32).reshape(-1, _EMB)
    gathered = _sc_gather(node_emb, nbr2d, path.astype(jnp.int32))
    return _tc_dense(gathered, W_pool, b_pool, Wk, Wr, b_lstm,
                     W1, b1, W2, b2, W3, b3)


# speedup vs baseline: 1.0978x; 1.0106x over previous
import functools

import jax
import jax.numpy as jnp
from jax import lax
from jax.experimental import pallas as pl
from jax.experimental.pallas import tpu as pltpu
from jax.experimental.pallas import tpu_sc as plsc

_EMB = 128
_K = 32
_STEP = 256
_NSTEP = 4
_PACK = 128 // _K
_NROWS = _NSTEP * _K + 16


def _lane_gather(vec, idx):
    return lax.gather(
        vec, idx[:, None],
        lax.GatherDimensionNumbers(
            offset_dims=(), collapsed_slice_dims=(0,), start_index_map=(0,)),
        (1,), mode=lax.GatherScatterMode.PROMISE_IN_BOUNDS)


def _eq_mask(a, b):
    d = a - b
    return 1 + ((d | -d) >> 31)


def _sc_gather(node_emb, neighbors2d, path_i32):
    mesh = plsc.VectorSubcoreMesh(core_axis_name="c", subcore_axis_name="s",
                                  num_cores=1)

    @functools.partial(
        pl.kernel,
        out_type=jax.ShapeDtypeStruct((_NROWS, _EMB), jnp.float32),
        mesh=mesh,
        scratch_types=[
            pltpu.VMEM((16,), jnp.int32),
            pltpu.VMEM((16, _EMB), jnp.int32),
            pltpu.VMEM((16, _EMB), jnp.float32),
            pltpu.SemaphoreType.DMA,
            pltpu.SemaphoreType.DMA,
        ],
    )
    def gather_kernel(emb_hbm, nbrtab_hbm, path_hbm, out_hbm,
                      path_v, nbrrows_v, emb_v, sem_idx, sem_emb):
        wid = lax.axis_index("c") * 16 + lax.axis_index("s")

        @pl.when(wid <= 2 * _NSTEP)
        def _():
            cp_a = pltpu.async_copy(path_hbm.at[pl.ds(0, 8)],
                                    path_v.at[pl.ds(0, 8)], sem_idx)
            cp_b = pltpu.async_copy(path_hbm.at[pl.ds(8, 1)],
                                    path_v.at[pl.ds(8, 1)], sem_idx)
            cp_a.wait()
            cp_b.wait()
            lanes = lax.iota(jnp.int32, 16)
            roots = _lane_gather(path_v[...], jnp.minimum(lanes * 2 + 2, 8))
            w = jnp.minimum(wid >> 1, _NSTEP - 1)
            h = wid & 1
            rw = _lane_gather(roots, jnp.full((16,), w, jnp.int32))
            pltpu.async_copy(nbrtab_hbm.at[rw >> 2], nbrrows_v,
                             sem_idx).wait()
            bb = ((rw & (_PACK - 1)) << 1) + h
            ids = jnp.zeros((16,), jnp.int32)
            for t in range(8):
                ids = ids + nbrrows_v[0, pl.ds(16 * t, 16)] * _eq_mask(bb, t)
            m8 = _eq_mask(wid, 2 * _NSTEP)
            ids = ids * (1 - m8) + roots * m8
            pltpu.async_copy(emb_hbm.at[ids], emb_v, sem_emb).wait()
            pltpu.sync_copy(emb_v, out_hbm.at[pl.ds(wid * 16, 16)])

    return gather_kernel(node_emb, neighbors2d, path_i32)


def _dense_body(g_hbm, wp_hbm, bp_hbm, wk_hbm, wr_hbm, bl_hbm,
                w1_hbm, b1_hbm, w2_hbm, b2_hbm, w3_hbm, b3_hbm, out_ref,
                g_v, wp_v, bp_v, wk_v, wr_v, bl_v, w1_v, b1_v, w2_v, b2_v,
                w3_v, b3_v, sem_a, sem_k, sem_r, sem_1, sem_2, sem_3):
    f32 = jnp.float32

    def dot(a, b):
        return lax.dot_general(a, b, (((1,), (0,)), ((), ())),
                               preferred_element_type=f32,
                               precision=lax.Precision.DEFAULT)

    cp = pltpu.make_async_copy
    cps_a = [cp(g_hbm, g_v, sem_a), cp(wp_hbm, wp_v, sem_a),
             cp(bp_hbm, bp_v, sem_a)]
    for c_ in cps_a:
        c_.start()
    cps_k = [cp(wk_hbm, wk_v, sem_k), cp(bl_hbm, bl_v, sem_k)]
    for c_ in cps_k:
        c_.start()
    cp_r = cp(wr_hbm, wr_v, sem_r)
    cp_r.start()
    cps_1 = [cp(w1_hbm, w1_v, sem_1), cp(b1_hbm, b1_v, sem_1)]
    for c_ in cps_1:
        c_.start()
    cps_2 = [cp(w2_hbm, w2_v, sem_2), cp(b2_hbm, b2_v, sem_2)]
    for c_ in cps_2:
        c_.start()
    cps_3 = [cp(w3_hbm, w3_v, sem_3), cp(b3_hbm, b3_v, sem_3)]
    for c_ in cps_3:
        c_.start()

    for c_ in cps_a:
        c_.wait()
    wp = wp_v[...]
    nbr_p = dot(g_v[:_NSTEP * _K, :], wp[_EMB:])
    self_p = dot(g_v[_NSTEP * _K:_NSTEP * _K + 8, :], wp[:_EMB])
    bp = bp_v[...][None, :]

    sfs = []
    for w in range(_NSTEP):
        blk = nbr_p[_K * w:_K * (w + 1)] + self_p[w][None, :] + bp
        blk = jnp.maximum(blk, 0.0)
        sfs.append(jnp.max(blk, axis=0, keepdims=True))
    sf = jnp.concatenate(sfs, axis=0)

    for c_ in cps_k:
        c_.wait()
    pre = dot(sf, wk_v[...]) + bl_v[...][None, :]
    cp_r.wait()
    wr = wr_v[...]
    h = jnp.zeros((1, _STEP), f32)
    c = jnp.zeros((1, _STEP), f32)
    for i in range(_NSTEP):
        z = pre[i:i + 1] + dot(h, wr)
        zi = z[:, :_STEP]
        zf = z[:, _STEP:2 * _STEP]
        zc = z[:, 2 * _STEP:3 * _STEP]
        zo = z[:, 3 * _STEP:]
        c = jax.nn.sigmoid(zf) * c + jax.nn.sigmoid(zi) * jnp.tanh(zc)
        h = jax.nn.sigmoid(zo) * jnp.tanh(c)

    for c_ in cps_1:
        c_.wait()
    h1 = jnp.maximum(dot(h, w1_v[...]) + b1_v[...][None, :], 0.0)
    for c_ in cps_2:
        c_.wait()
    h2 = jnp.maximum(dot(h1, w2_v[...]) + b2_v[...][None, :], 0.0)
    for c_ in cps_3:
        c_.wait()
    logits = dot(h2, w3_v[...]) + b3_v[...][None, :]
    out_ref[...] = jax.nn.softmax(logits, axis=-1)[0]


def _tc_dense(gathered, W_pool, b_pool, Wk, Wr, b_lstm,
              W1, b1, W2, b2, W3, b3):
    any_spec = pl.BlockSpec(memory_space=pl.ANY)
    return pl.pallas_call(
        _dense_body,
        in_specs=[any_spec] * 12,
        out_shape=jax.ShapeDtypeStruct((2,), jnp.float32),
        scratch_shapes=[
            pltpu.VMEM((_NROWS, _EMB), jnp.float32),
            pltpu.VMEM((2 * _EMB, _STEP), jnp.float32),
            pltpu.VMEM((_STEP,), jnp.float32),
            pltpu.VMEM((_STEP, 4 * _STEP), jnp.float32),
            pltpu.VMEM((_STEP, 4 * _STEP), jnp.float32),
            pltpu.VMEM((4 * _STEP,), jnp.float32),
            pltpu.VMEM((_STEP, _STEP), jnp.float32),
            pltpu.VMEM((_STEP,), jnp.float32),
            pltpu.VMEM((_STEP, _STEP), jnp.float32),
            pltpu.VMEM((_STEP,), jnp.float32),
            pltpu.VMEM((_STEP, 2), jnp.float32),
            pltpu.VMEM((2,), jnp.float32),
            pltpu.SemaphoreType.DMA,
            pltpu.SemaphoreType.DMA,
            pltpu.SemaphoreType.DMA,
            pltpu.SemaphoreType.DMA,
            pltpu.SemaphoreType.DMA,
            pltpu.SemaphoreType.DMA,
        ],
    )(gathered, W_pool, b_pool, Wk, Wr, b_lstm, W1, b1, W2, b2, W3, b3)


def kernel(node_emb, neighbors, path, W_pool, b_pool, Wk, Wr, b_lstm,
           W1, b1, W2, b2, W3, b3):
    nbr2d = neighbors.astype(jnp.int32).reshape(-1, _EMB)
    gathered = _sc_gather(node_emb, nbr2d, path.astype(jnp.int32))
    return _tc_dense(gathered, W_pool, b_pool, Wk, Wr, b_lstm,
                     W1, b1, W2, b2, W3, b3)


# speedup vs baseline: 1.1013x; 1.0032x over previous
import functools

import jax
import jax.numpy as jnp
from jax import lax
from jax.experimental import pallas as pl
from jax.experimental.pallas import tpu as pltpu
from jax.experimental.pallas import tpu_sc as plsc

_EMB = 128
_K = 32
_STEP = 256
_NSTEP = 4
_PACK = 128 // _K
_NROWS = _NSTEP * _K + 16


def _lane_gather(vec, idx):
    return lax.gather(
        vec, idx[:, None],
        lax.GatherDimensionNumbers(
            offset_dims=(), collapsed_slice_dims=(0,), start_index_map=(0,)),
        (1,), mode=lax.GatherScatterMode.PROMISE_IN_BOUNDS)


def _eq_mask(a, b):
    d = a - b
    return 1 + ((d | -d) >> 31)


def _sc_gather(node_emb, neighbors2d, path_i32):
    mesh = plsc.VectorSubcoreMesh(core_axis_name="c", subcore_axis_name="s",
                                  num_cores=1, num_subcores=2 * _NSTEP + 1)

    @functools.partial(
        pl.kernel,
        out_type=jax.ShapeDtypeStruct((_NROWS, _EMB), jnp.float32),
        mesh=mesh,
        scratch_types=[
            pltpu.VMEM((16,), jnp.int32),
            pltpu.VMEM((16, _EMB), jnp.int32),
            pltpu.VMEM((16, _EMB), jnp.float32),
            pltpu.SemaphoreType.DMA,
            pltpu.SemaphoreType.DMA,
        ],
    )
    def gather_kernel(emb_hbm, nbrtab_hbm, path_hbm, out_hbm,
                      path_v, nbrrows_v, emb_v, sem_idx, sem_emb):
        wid = lax.axis_index("c") * 16 + lax.axis_index("s")

        @pl.when(wid <= 2 * _NSTEP)
        def _():
            cp_a = pltpu.async_copy(path_hbm.at[pl.ds(0, 8)],
                                    path_v.at[pl.ds(0, 8)], sem_idx)
            cp_b = pltpu.async_copy(path_hbm.at[pl.ds(8, 1)],
                                    path_v.at[pl.ds(8, 1)], sem_idx)
            cp_a.wait()
            cp_b.wait()
            lanes = lax.iota(jnp.int32, 16)
            roots = _lane_gather(path_v[...], jnp.minimum(lanes * 2 + 2, 8))
            w = jnp.minimum(wid >> 1, _NSTEP - 1)
            h = wid & 1
            rw = _lane_gather(roots, jnp.full((16,), w, jnp.int32))
            pltpu.async_copy(nbrtab_hbm.at[rw >> 2], nbrrows_v,
                             sem_idx).wait()
            bb = ((rw & (_PACK - 1)) << 1) + h
            ids = jnp.zeros((16,), jnp.int32)
            for t in range(8):
                ids = ids + nbrrows_v[0, pl.ds(16 * t, 16)] * _eq_mask(bb, t)
            m8 = _eq_mask(wid, 2 * _NSTEP)
            ids = ids * (1 - m8) + roots * m8
            pltpu.async_copy(emb_hbm.at[ids], emb_v, sem_emb).wait()
            pltpu.sync_copy(emb_v, out_hbm.at[pl.ds(wid * 16, 16)])

    return gather_kernel(node_emb, neighbors2d, path_i32)


def _dense_body(g_hbm, wp_hbm, bp_hbm, wk_hbm, wr_hbm, bl_hbm,
                w1_hbm, b1_hbm, w2_hbm, b2_hbm, w3_hbm, b3_hbm, out_ref,
                g_v, wp_v, bp_v, wk_v, wr_v, bl_v, w1_v, b1_v, w2_v, b2_v,
                w3_v, b3_v, sem_a, sem_k, sem_r, sem_1, sem_2, sem_3):
    f32 = jnp.float32

    def dot(a, b):
        return lax.dot_general(a, b, (((1,), (0,)), ((), ())),
                               preferred_element_type=f32,
                               precision=lax.Precision.DEFAULT)

    cp = pltpu.make_async_copy
    cps_a = [cp(g_hbm, g_v, sem_a), cp(wp_hbm, wp_v, sem_a),
             cp(bp_hbm, bp_v, sem_a)]
    for c_ in cps_a:
        c_.start()
    cps_k = [cp(wk_hbm, wk_v, sem_k), cp(bl_hbm, bl_v, sem_k)]
    for c_ in cps_k:
        c_.start()
    cp_r = cp(wr_hbm, wr_v, sem_r)
    cp_r.start()
    cps_1 = [cp(w1_hbm, w1_v, sem_1), cp(b1_hbm, b1_v, sem_1)]
    for c_ in cps_1:
        c_.start()
    cps_2 = [cp(w2_hbm, w2_v, sem_2), cp(b2_hbm, b2_v, sem_2)]
    for c_ in cps_2:
        c_.start()
    cps_3 = [cp(w3_hbm, w3_v, sem_3), cp(b3_hbm, b3_v, sem_3)]
    for c_ in cps_3:
        c_.start()

    for c_ in cps_a:
        c_.wait()
    wp = wp_v[...]
    nbr_p = dot(g_v[:_NSTEP * _K, :], wp[_EMB:])
    self_p = dot(g_v[_NSTEP * _K:_NSTEP * _K + 8, :], wp[:_EMB])
    bp = bp_v[...][None, :]

    sfs = []
    for w in range(_NSTEP):
        blk = nbr_p[_K * w:_K * (w + 1)] + self_p[w][None, :] + bp
        blk = jnp.maximum(blk, 0.0)
        sfs.append(jnp.max(blk, axis=0, keepdims=True))
    sf = jnp.concatenate(sfs, axis=0)

    for c_ in cps_k:
        c_.wait()
    pre = dot(sf, wk_v[...]) + bl_v[...][None, :]
    cp_r.wait()
    wr = wr_v[...]
    h = jnp.zeros((1, _STEP), f32)
    c = jnp.zeros((1, _STEP), f32)
    for i in range(_NSTEP):
        z = pre[i:i + 1] + dot(h, wr)
        zi = z[:, :_STEP]
        zf = z[:, _STEP:2 * _STEP]
        zc = z[:, 2 * _STEP:3 * _STEP]
        zo = z[:, 3 * _STEP:]
        c = jax.nn.sigmoid(zf) * c + jax.nn.sigmoid(zi) * jnp.tanh(zc)
        h = jax.nn.sigmoid(zo) * jnp.tanh(c)

    for c_ in cps_1:
        c_.wait()
    h1 = jnp.maximum(dot(h, w1_v[...]) + b1_v[...][None, :], 0.0)
    for c_ in cps_2:
        c_.wait()
    h2 = jnp.maximum(dot(h1, w2_v[...]) + b2_v[...][None, :], 0.0)
    for c_ in cps_3:
        c_.wait()
    logits = dot(h2, w3_v[...]) + b3_v[...][None, :]
    out_ref[...] = jax.nn.softmax(logits, axis=-1)[0]


def _tc_dense(gathered, W_pool, b_pool, Wk, Wr, b_lstm,
              W1, b1, W2, b2, W3, b3):
    any_spec = pl.BlockSpec(memory_space=pl.ANY)
    return pl.pallas_call(
        _dense_body,
        in_specs=[any_spec] * 12,
        out_shape=jax.ShapeDtypeStruct((2,), jnp.float32),
        scratch_shapes=[
            pltpu.VMEM((_NROWS, _EMB), jnp.float32),
            pltpu.VMEM((2 * _EMB, _STEP), jnp.float32),
            pltpu.VMEM((_STEP,), jnp.float32),
            pltpu.VMEM((_STEP, 4 * _STEP), jnp.float32),
            pltpu.VMEM((_STEP, 4 * _STEP), jnp.float32),
            pltpu.VMEM((4 * _STEP,), jnp.float32),
            pltpu.VMEM((_STEP, _STEP), jnp.float32),
            pltpu.VMEM((_STEP,), jnp.float32),
            pltpu.VMEM((_STEP, _STEP), jnp.float32),
            pltpu.VMEM((_STEP,), jnp.float32),
            pltpu.VMEM((_STEP, 2), jnp.float32),
            pltpu.VMEM((2,), jnp.float32),
            pltpu.SemaphoreType.DMA,
            pltpu.SemaphoreType.DMA,
            pltpu.SemaphoreType.DMA,
            pltpu.SemaphoreType.DMA,
            pltpu.SemaphoreType.DMA,
            pltpu.SemaphoreType.DMA,
        ],
    )(gathered, W_pool, b_pool, Wk, Wr, b_lstm, W1, b1, W2, b2, W3, b3)


def kernel(node_emb, neighbors, path, W_pool, b_pool, Wk, Wr, b_lstm,
           W1, b1, W2, b2, W3, b3):
    nbr2d = neighbors.astype(jnp.int32).reshape(-1, _EMB)
    gathered = _sc_gather(node_emb, nbr2d, path.astype(jnp.int32))
    return _tc_dense(gathered, W_pool, b_pool, Wk, Wr, b_lstm,
                     W1, b1, W2, b2, W3, b3)
